# baseline (speedup 1.0000x reference)
import functools

import jax
import jax.numpy as jnp
from jax import lax
from jax.experimental import pallas as pl
from jax.experimental.pallas import tpu as pltpu
from jax.experimental.pallas import tpu_sc as plsc

B = 64
L = 1024
D = 768
LANES = 16

_NC = 2
_NS = 16
_NW = _NC * _NS
_P = L // _NW
_VECS = D // LANES
_NBUF = 4


def _add_pos(row_v, pos_v):
    def add_row(r, c2):
        for c in range(_VECS):
            sl = pl.ds(c * LANES, LANES)
            plsc.addupdate(row_v.at[r, sl], pos_v[r, sl])
        return c2

    lax.fori_loop(0, _P, add_row, 0, unroll=4)


def _emb_kernel(x_hbm, tok_hbm, pos_hbm, out_hbm,
                pos_v, idx_v, row_0, row_1, row_2, row_3,
                isem, psem,
                gsem_0, gsem_1, gsem_2, gsem_3,
                wsem_0, wsem_1, wsem_2, wsem_3):
    wid = lax.axis_index("s") * _NC + lax.axis_index("c")
    pbase = wid * _P

    rows = (row_0, row_1, row_2, row_3)
    gsems = (gsem_0, gsem_1, gsem_2, gsem_3)
    wsems = (wsem_0, wsem_1, wsem_2, wsem_3)

    def gather_start(b, slot):
        pltpu.async_copy(tok_hbm.at[idx_v.at[pl.ds(b * _P, _P)]],
                         rows[slot], gsems[slot])

    def gather_wait(b, slot):
        pltpu.make_async_copy(tok_hbm.at[idx_v.at[pl.ds(b * _P, _P)]],
                              rows[slot], gsems[slot]).wait()

    def write_start(b, slot):
        dst = out_hbm.at[pl.ds(b * L + pbase, _P)]
        pltpu.async_copy(rows[slot], dst, wsems[slot])

    def write_wait(b, slot):
        dst = out_hbm.at[pl.ds(b * L + pbase, _P)]
        pltpu.make_async_copy(rows[slot], dst, wsems[slot]).wait()

    idx_cp = pltpu.async_copy(x_hbm.at[pl.ds(wid * B * _P, B * _P)],
                              idx_v, isem)
    idx_cp.wait()
    gather_start(0, 0)
    gather_start(1, 1)
    pos_cp = pltpu.async_copy(pos_hbm.at[pl.ds(pbase, _P)], pos_v, psem)
    pos_cp.wait()

    def body(i, carry):
        for k in range(_NBUF):
            b = 4 * i + k
            s2 = (k + 2) % _NBUF

            if k < 2:
                @pl.when(i > 0)
                def _():
                    write_wait(b - 2, s2)

                gather_start(b + 2, s2)
            else:
                @pl.when(i < B // _NBUF - 1)
                def _():
                    write_wait(b - 2, s2)
                    gather_start(b + 2, s2)

            gather_wait(b, k)
            _add_pos(rows[k], pos_v)
            write_start(b, k)
        return carry

    lax.fori_loop(0, B // _NBUF, body, 0)
    for k in range(_NBUF):
        write_wait(B - _NBUF + k, k)


@jax.jit
def kernel(x, token_table, pos_table):
    xp = x.astype(jnp.int32).reshape(B, _NW, _P).transpose(1, 0, 2).reshape(-1)
    mesh = plsc.VectorSubcoreMesh(core_axis_name="c", subcore_axis_name="s")
    out = pl.kernel(
        _emb_kernel,
        out_type=jax.ShapeDtypeStruct((B * L, D), jnp.float32),
        mesh=mesh,
        scratch_types=[
            pltpu.VMEM((_P, D), jnp.float32),
            pltpu.VMEM((B * _P,), jnp.int32),
            pltpu.VMEM((_P, D), jnp.float32),
            pltpu.VMEM((_P, D), jnp.float32),
            pltpu.VMEM((_P, D), jnp.float32),
            pltpu.VMEM((_P, D), jnp.float32),
            pltpu.SemaphoreType.DMA,
            pltpu.SemaphoreType.DMA,
            pltpu.SemaphoreType.DMA,
            pltpu.SemaphoreType.DMA,
            pltpu.SemaphoreType.DMA,
            pltpu.SemaphoreType.DMA,
            pltpu.SemaphoreType.DMA,
            pltpu.SemaphoreType.DMA,
            pltpu.SemaphoreType.DMA,
            pltpu.SemaphoreType.DMA,
        ],
    )(xp, token_table, pos_table)
    return out.reshape(B, L, D)


# speedup vs baseline: 1.0081x; 1.0081x over previous
import functools

import jax
import jax.numpy as jnp
from jax import lax
from jax.experimental import pallas as pl
from jax.experimental.pallas import tpu as pltpu
from jax.experimental.pallas import tpu_sc as plsc

B = 64
L = 1024
D = 768
LANES = 16

_NC = 2
_NS = 16
_NW = _NC * _NS
_P = L // _NW
_VECS = D // LANES
_NBUF = 4


def _add_pos(row_v, pos_v):
    def add_row(r, c2):
        for c in range(_VECS):
            sl = pl.ds(c * LANES, LANES)
            plsc.addupdate(row_v.at[r, sl], pos_v[r, sl])
        return c2

    lax.fori_loop(0, _P, add_row, 0, unroll=2)


def _emb_kernel(x_hbm, tok_hbm, pos_hbm, out_hbm,
                pos_v, idx_v, row_0, row_1, row_2, row_3,
                isem, psem,
                gsem_0, gsem_1, gsem_2, gsem_3,
                wsem_0, wsem_1, wsem_2, wsem_3):
    wid = lax.axis_index("s") * _NC + lax.axis_index("c")
    pbase = wid * _P

    rows = (row_0, row_1, row_2, row_3)
    gsems = (gsem_0, gsem_1, gsem_2, gsem_3)
    wsems = (wsem_0, wsem_1, wsem_2, wsem_3)

    def gather_start(b, slot):
        pltpu.async_copy(tok_hbm.at[idx_v.at[pl.ds(b * _P, _P)]],
                         rows[slot], gsems[slot])

    def gather_wait(b, slot):
        pltpu.make_async_copy(tok_hbm.at[idx_v.at[pl.ds(b * _P, _P)]],
                              rows[slot], gsems[slot]).wait()

    def write_start(b, slot):
        dst = out_hbm.at[pl.ds(b * L + pbase, _P)]
        pltpu.async_copy(rows[slot], dst, wsems[slot])

    def write_wait(b, slot):
        dst = out_hbm.at[pl.ds(b * L + pbase, _P)]
        pltpu.make_async_copy(rows[slot], dst, wsems[slot]).wait()

    idx_cp = pltpu.async_copy(x_hbm.at[pl.ds(wid * B * _P, B * _P)],
                              idx_v, isem)
    idx_cp.wait()
    gather_start(0, 0)
    gather_start(1, 1)
    pos_cp = pltpu.async_copy(pos_hbm.at[pl.ds(pbase, _P)], pos_v, psem)
    pos_cp.wait()

    def body(i, carry):
        for k in range(_NBUF):
            b = 4 * i + k
            s2 = (k + 2) % _NBUF

            if k < 2:
                @pl.when(i > 0)
                def _():
                    write_wait(b - 2, s2)

                gather_start(b + 2, s2)
            else:
                @pl.when(i < B // _NBUF - 1)
                def _():
                    write_wait(b - 2, s2)
                    gather_start(b + 2, s2)

            gather_wait(b, k)
            _add_pos(rows[k], pos_v)
            write_start(b, k)
        return carry

    lax.fori_loop(0, B // _NBUF, body, 0)
    for k in range(_NBUF):
        write_wait(B - _NBUF + k, k)


@jax.jit
def kernel(x, token_table, pos_table):
    xp = x.astype(jnp.int32).reshape(B, _NW, _P).transpose(1, 0, 2).reshape(-1)
    mesh = plsc.VectorSubcoreMesh(core_axis_name="c", subcore_axis_name="s")
    out = pl.kernel(
        _emb_kernel,
        out_type=jax.ShapeDtypeStruct((B * L, D), jnp.float32),
        mesh=mesh,
        scratch_types=[
            pltpu.VMEM((_P, D), jnp.float32),
            pltpu.VMEM((B * _P,), jnp.int32),
            pltpu.VMEM((_P, D), jnp.float32),
            pltpu.VMEM((_P, D), jnp.float32),
            pltpu.VMEM((_P, D), jnp.float32),
            pltpu.VMEM((_P, D), jnp.float32),
            pltpu.SemaphoreType.DMA,
            pltpu.SemaphoreType.DMA,
            pltpu.SemaphoreType.DMA,
            pltpu.SemaphoreType.DMA,
            pltpu.SemaphoreType.DMA,
            pltpu.SemaphoreType.DMA,
            pltpu.SemaphoreType.DMA,
            pltpu.SemaphoreType.DMA,
            pltpu.SemaphoreType.DMA,
            pltpu.SemaphoreType.DMA,
        ],
    )(xp, token_table, pos_table)
    return out.reshape(B, L, D)
